# baseline (speedup 1.0000x reference)
import functools

import jax
import jax.numpy as jnp
from jax import lax
from jax.experimental import pallas as pl
from jax.experimental.pallas import tpu as pltpu
from jax.experimental.pallas import tpu_sc as plsc

_NUM_CORES = 2
_NUM_SUBCORES = 16
_NUM_WORKERS = _NUM_CORES * _NUM_SUBCORES
_CHUNK = 400
_LANES = 16


@functools.partial(jax.jit, static_argnames=("n_chunks", "d_word", "d_pos"))
def _embed(word_i, pos1_i, pos2_i, word_table2, pos1_table, pos2_table,
           n_chunks, d_word, d_pos):
    n_total = _NUM_WORKERS * n_chunks * _CHUNK
    d_out = d_word + 2 * d_pos
    mesh = plsc.VectorSubcoreMesh(core_axis_name="c", subcore_axis_name="s")

    @functools.partial(
        pl.kernel,
        mesh=mesh,
        compiler_params=pltpu.CompilerParams(use_tc_tiling_on_sc=False,
                                             needs_layout_passes=False),
        out_type=jax.ShapeDtypeStruct((n_total, d_out), jnp.float32),
        scratch_types=[
            pltpu.VMEM((n_chunks, _CHUNK), jnp.int32),
            pltpu.VMEM((n_chunks, _CHUNK), jnp.int32),
            pltpu.VMEM((n_chunks, _CHUNK), jnp.int32),
            pltpu.VMEM((_CHUNK,), jnp.int32),
            pltpu.VMEM((_CHUNK, 2 * d_word), jnp.float32),
            pltpu.VMEM((_CHUNK, d_word), jnp.float32),
            pltpu.VMEM((_CHUNK, d_pos), jnp.float32),
            pltpu.VMEM((_CHUNK, d_pos), jnp.float32),
            pltpu.SemaphoreType.DMA,
        ],
    )
    def emb_kernel(w_hbm, p1_hbm, p2_hbm, wt_hbm, p1t_hbm, p2t_hbm, out_hbm,
                   widx, p1idx, p2idx, hidx, wbuf, wsel, p1buf, p2buf, sem):
        wid = lax.axis_index("s") * _NUM_CORES + lax.axis_index("c")
        pltpu.sync_copy(w_hbm.at[wid], widx)
        pltpu.sync_copy(p1_hbm.at[wid], p1idx)
        pltpu.sync_copy(p2_hbm.at[wid], p2idx)
        base0 = wid * (n_chunks * _CHUNK)
        lane = lax.iota(jnp.int32, _LANES)
        n_grp = _CHUNK // _LANES

        def body(j, carry):
            def halve(g, c):
                v = widx[j, pl.ds(g * _LANES, _LANES)]
                hidx[pl.ds(g * _LANES, _LANES)] = lax.shift_right_logical(v, 1)
                return c

            lax.fori_loop(0, n_grp, halve, 0)

            cw = pltpu.async_copy(wt_hbm.at[hidx], wbuf, sem)
            c1 = pltpu.async_copy(p1t_hbm.at[p1idx.at[j]], p1buf, sem)
            c2 = pltpu.async_copy(p2t_hbm.at[p2idx.at[j]], p2buf, sem)
            cw.wait()
            c1.wait()
            c2.wait()

            def select(g, c):
                t = g * _LANES + lane
                v = widx[j, pl.ds(g * _LANES, _LANES)]
                pcol = lax.bitwise_and(v, 1) * d_word
                for e in range(d_word):
                    val = plsc.load_gather(wbuf, [t, pcol + e])
                    plsc.store_scatter(wsel, [t, lane * 0 + e], val)
                return c

            lax.fori_loop(0, n_grp, select, 0)

            base = base0 + j * _CHUNK
            pltpu.sync_copy(wsel, out_hbm.at[pl.ds(base, _CHUNK),
                                             pl.ds(0, d_word)])
            pltpu.sync_copy(p1buf, out_hbm.at[pl.ds(base, _CHUNK),
                                              pl.ds(d_word, d_pos)])
            pltpu.sync_copy(p2buf, out_hbm.at[pl.ds(base, _CHUNK),
                                              pl.ds(d_word + d_pos, d_pos)])
            return carry

        lax.fori_loop(0, n_chunks, body, 0)

    return emb_kernel(word_i, pos1_i, pos2_i,
                      word_table2, pos1_table, pos2_table)


def kernel(word, pos1, pos2, word_table, pos1_table, pos2_table):
    b, l = word.shape
    d_word = word_table.shape[1]
    d_pos = pos1_table.shape[1]
    n = b * l
    assert n % (_NUM_WORKERS * _CHUNK) == 0
    n_chunks = n // (_NUM_WORKERS * _CHUNK)

    shape = (_NUM_WORKERS, n_chunks, _CHUNK)
    word_i = word.reshape(shape).astype(jnp.int32)
    pos1_i = pos1.reshape(shape).astype(jnp.int32)
    pos2_i = pos2.reshape(shape).astype(jnp.int32)
    vocab = word_table.shape[0]
    word_table2 = word_table.reshape(vocab // 2, 2 * d_word)

    out = _embed(word_i, pos1_i, pos2_i,
                 word_table2, pos1_table, pos2_table,
                 n_chunks, d_word, d_pos)
    return out.reshape(b, l, d_word + 2 * d_pos)


# speedup vs baseline: 1.7020x; 1.7020x over previous
import functools

import jax
import jax.numpy as jnp
from jax import lax
from jax.experimental import pallas as pl
from jax.experimental.pallas import tpu as pltpu
from jax.experimental.pallas import tpu_sc as plsc

_NUM_CORES = 2
_NUM_SUBCORES = 16
_NUM_WORKERS = _NUM_CORES * _NUM_SUBCORES
_CHUNK = 800
_D_OUT = 128


@functools.partial(jax.jit, static_argnames=("n_chunks", "d_word", "d_pos"))
def _embed(word_i, pos1_i, pos2_i, word_table, pos1_table, pos2_table,
           n_chunks, d_word, d_pos):
    n_total = _NUM_WORKERS * n_chunks * _CHUNK
    mesh = plsc.VectorSubcoreMesh(core_axis_name="c", subcore_axis_name="s")

    @functools.partial(
        pl.kernel,
        mesh=mesh,
        compiler_params=pltpu.CompilerParams(use_tc_tiling_on_sc=False),
        out_type=jax.ShapeDtypeStruct((n_total, _D_OUT), jnp.float32),
        scratch_types=[
            pltpu.VMEM((n_chunks, _CHUNK), jnp.int32),
            pltpu.VMEM((n_chunks, _CHUNK), jnp.int32),
            pltpu.VMEM((n_chunks, _CHUNK), jnp.int32),
            pltpu.VMEM((_CHUNK, 64), jnp.float32),
            pltpu.VMEM((_CHUNK, 16), jnp.float32),
            pltpu.VMEM((_CHUNK, 16), jnp.float32),
            pltpu.SemaphoreType.DMA,
        ],
    )
    def emb_kernel(w_hbm, p1_hbm, p2_hbm, wt_hbm, p1t_hbm, p2t_hbm, out_hbm,
                   widx, p1idx, p2idx, wbuf, p1buf, p2buf, sem):
        wid = lax.axis_index("s") * _NUM_CORES + lax.axis_index("c")
        pltpu.sync_copy(w_hbm.at[wid], widx)
        pltpu.sync_copy(p1_hbm.at[wid], p1idx)
        pltpu.sync_copy(p2_hbm.at[wid], p2idx)
        base0 = wid * (n_chunks * _CHUNK)

        def body(j, carry):
            cw = pltpu.async_copy(wt_hbm.at[widx.at[j]], wbuf, sem)
            c1 = pltpu.async_copy(p1t_hbm.at[p1idx.at[j]], p1buf, sem)
            c2 = pltpu.async_copy(p2t_hbm.at[p2idx.at[j]], p2buf, sem)
            cw.wait()
            c1.wait()
            c2.wait()
            base = base0 + j * _CHUNK
            pltpu.sync_copy(wbuf, out_hbm.at[pl.ds(base, _CHUNK),
                                             pl.ds(0, d_word)])
            pltpu.sync_copy(p1buf, out_hbm.at[pl.ds(base, _CHUNK),
                                              pl.ds(d_word, d_pos)])
            pltpu.sync_copy(p2buf, out_hbm.at[pl.ds(base, _CHUNK),
                                              pl.ds(d_word + d_pos, d_pos)])
            return carry

        lax.fori_loop(0, n_chunks, body, 0)

    return emb_kernel(word_i, pos1_i, pos2_i,
                      word_table, pos1_table, pos2_table)


def kernel(word, pos1, pos2, word_table, pos1_table, pos2_table):
    b, l = word.shape
    d_word = word_table.shape[1]
    d_pos = pos1_table.shape[1]
    n = b * l
    assert n % (_NUM_WORKERS * _CHUNK) == 0
    n_chunks = n // (_NUM_WORKERS * _CHUNK)

    shape = (_NUM_WORKERS, n_chunks, _CHUNK)
    word_i = word.reshape(shape).astype(jnp.int32)
    pos1_i = pos1.reshape(shape).astype(jnp.int32)
    pos2_i = pos2.reshape(shape).astype(jnp.int32)

    out = _embed(word_i, pos1_i, pos2_i,
                 word_table, pos1_table, pos2_table,
                 n_chunks, d_word, d_pos)
    return out[:, :d_word + 2 * d_pos].reshape(b, l, d_word + 2 * d_pos)


# speedup vs baseline: 1.7809x; 1.0464x over previous
import functools

import jax
import jax.numpy as jnp
from jax import lax
from jax.experimental import pallas as pl
from jax.experimental.pallas import tpu as pltpu
from jax.experimental.pallas import tpu_sc as plsc

_NUM_CORES = 2
_NUM_SUBCORES = 16
_NUM_WORKERS = _NUM_CORES * _NUM_SUBCORES
_CHUNK = 400
_D_OUT = 128


@functools.partial(jax.jit, static_argnames=("n_chunks", "d_word", "d_pos"))
def _embed(word_i, pos1_i, pos2_i, word_table, pos1_table, pos2_table,
           n_chunks, d_word, d_pos):
    n_total = _NUM_WORKERS * n_chunks * _CHUNK
    mesh = plsc.VectorSubcoreMesh(core_axis_name="c", subcore_axis_name="s")

    @functools.partial(
        pl.kernel,
        mesh=mesh,
        compiler_params=pltpu.CompilerParams(use_tc_tiling_on_sc=False),
        out_type=jax.ShapeDtypeStruct((n_total, _D_OUT), jnp.float32),
        scratch_types=[
            pltpu.VMEM((n_chunks, _CHUNK), jnp.int32),
            pltpu.VMEM((n_chunks, _CHUNK), jnp.int32),
            pltpu.VMEM((n_chunks, _CHUNK), jnp.int32),
            pltpu.VMEM((_CHUNK, _D_OUT), jnp.float32),
            pltpu.VMEM((_CHUNK, 16), jnp.float32),
            pltpu.VMEM((_CHUNK, 16), jnp.float32),
            pltpu.SemaphoreType.DMA,
        ],
    )
    def emb_kernel(w_hbm, p1_hbm, p2_hbm, wt_hbm, p1t_hbm, p2t_hbm, out_hbm,
                   widx, p1idx, p2idx, wbuf, p1buf, p2buf, sem):
        wid = lax.axis_index("s") * _NUM_CORES + lax.axis_index("c")
        pltpu.sync_copy(w_hbm.at[wid], widx)
        pltpu.sync_copy(p1_hbm.at[wid], p1idx)
        pltpu.sync_copy(p2_hbm.at[wid], p2idx)
        base0 = wid * (n_chunks * _CHUNK)

        def body(j, carry):
            cw = pltpu.async_copy(wt_hbm.at[widx.at[j]], wbuf, sem)
            c1 = pltpu.async_copy(p1t_hbm.at[p1idx.at[j]], p1buf, sem)
            c2 = pltpu.async_copy(p2t_hbm.at[p2idx.at[j]], p2buf, sem)
            cw.wait()
            c1.wait()
            c2.wait()
            base = base0 + j * _CHUNK
            pltpu.sync_copy(wbuf, out_hbm.at[pl.ds(base, _CHUNK)])
            pltpu.sync_copy(p1buf, out_hbm.at[pl.ds(base, _CHUNK),
                                              pl.ds(d_word, d_pos)])
            pltpu.sync_copy(p2buf, out_hbm.at[pl.ds(base, _CHUNK),
                                              pl.ds(d_word + d_pos, d_pos)])
            return carry

        lax.fori_loop(0, n_chunks, body, 0)

    return emb_kernel(word_i, pos1_i, pos2_i,
                      word_table, pos1_table, pos2_table)


def kernel(word, pos1, pos2, word_table, pos1_table, pos2_table):
    b, l = word.shape
    d_word = word_table.shape[1]
    d_pos = pos1_table.shape[1]
    n = b * l
    assert n % (_NUM_WORKERS * _CHUNK) == 0
    n_chunks = n // (_NUM_WORKERS * _CHUNK)

    shape = (_NUM_WORKERS, n_chunks, _CHUNK)
    word_i = word.reshape(shape).astype(jnp.int32)
    pos1_i = pos1.reshape(shape).astype(jnp.int32)
    pos2_i = pos2.reshape(shape).astype(jnp.int32)
    vocab = word_table.shape[0]
    word_table128 = jnp.concatenate(
        [word_table,
         jnp.zeros((vocab, _D_OUT - d_word), jnp.float32)], axis=1)

    out = _embed(word_i, pos1_i, pos2_i,
                 word_table128, pos1_table, pos2_table,
                 n_chunks, d_word, d_pos)
    return out[:, :d_word + 2 * d_pos].reshape(b, l, d_word + 2 * d_pos)


# speedup vs baseline: 1.8109x; 1.0169x over previous
import functools

import jax
import jax.numpy as jnp
from jax import lax
from jax.experimental import pallas as pl
from jax.experimental.pallas import tpu as pltpu
from jax.experimental.pallas import tpu_sc as plsc

_NUM_CORES = 2
_NUM_SUBCORES = 16
_NUM_WORKERS = _NUM_CORES * _NUM_SUBCORES
_CHUNK = 200
_D_OUT = 128


@functools.partial(jax.jit, static_argnames=("n_chunks", "d_word", "d_pos"))
def _embed(word_i, pos1_i, pos2_i, word_table, pos1_table, pos2_table,
           n_chunks, d_word, d_pos):
    n_total = _NUM_WORKERS * n_chunks * _CHUNK
    mesh = plsc.VectorSubcoreMesh(core_axis_name="c", subcore_axis_name="s")

    @functools.partial(
        pl.kernel,
        mesh=mesh,
        compiler_params=pltpu.CompilerParams(use_tc_tiling_on_sc=False),
        out_type=jax.ShapeDtypeStruct((n_total, _D_OUT), jnp.float32),
        scratch_types=[
            pltpu.VMEM((n_chunks, _CHUNK), jnp.int32),
            pltpu.VMEM((n_chunks, _CHUNK), jnp.int32),
            pltpu.VMEM((n_chunks, _CHUNK), jnp.int32),
            pltpu.VMEM((_CHUNK, _D_OUT), jnp.float32),
            pltpu.VMEM((_CHUNK, _D_OUT), jnp.float32),
            pltpu.VMEM((_CHUNK, 16), jnp.float32),
            pltpu.VMEM((_CHUNK, 16), jnp.float32),
            pltpu.VMEM((_CHUNK, 16), jnp.float32),
            pltpu.VMEM((_CHUNK, 16), jnp.float32),
            pltpu.SemaphoreType.DMA,
            pltpu.SemaphoreType.DMA,
            pltpu.SemaphoreType.DMA,
            pltpu.SemaphoreType.DMA,
        ],
    )
    def emb_kernel(w_hbm, p1_hbm, p2_hbm, wt_hbm, p1t_hbm, p2t_hbm, out_hbm,
                   widx, p1idx, p2idx, wbuf0, wbuf1, p1b0, p1b1, p2b0, p2b1,
                   gsem0, gsem1, wsem0, wsem1):
        wid = lax.axis_index("s") * _NUM_CORES + lax.axis_index("c")
        pltpu.sync_copy(w_hbm.at[wid], widx)
        pltpu.sync_copy(p1_hbm.at[wid], p1idx)
        pltpu.sync_copy(p2_hbm.at[wid], p2idx)
        base0 = wid * (n_chunks * _CHUNK)
        bufs = ((wbuf0, p1b0, p2b0, gsem0, wsem0),
                (wbuf1, p1b1, p2b1, gsem1, wsem1))

        def fire_gathers(j, s):
            wb, p1b, p2b, gs, _ = bufs[s]
            pltpu.async_copy(wt_hbm.at[widx.at[j]], wb, gs)
            pltpu.async_copy(p1t_hbm.at[p1idx.at[j]], p1b, gs)
            pltpu.async_copy(p2t_hbm.at[p2idx.at[j]], p2b, gs)

        def wait_gathers(s):
            wb, p1b, p2b, gs, _ = bufs[s]
            pltpu.make_async_copy(wt_hbm.at[pl.ds(0, _CHUNK)], wb, gs).wait()
            pltpu.make_async_copy(p1t_hbm.at[pl.ds(0, _CHUNK)], p1b, gs).wait()
            pltpu.make_async_copy(p2t_hbm.at[pl.ds(0, _CHUNK)], p2b, gs).wait()

        def fire_writes(j, s):
            wb, p1b, p2b, _, ws = bufs[s]
            base = base0 + j * _CHUNK
            rows = out_hbm.at[pl.ds(base, _CHUNK), pl.ds(0, d_word)]
            pltpu.async_copy(wb.at[:, pl.ds(0, d_word)], rows, ws)
            pltpu.async_copy(
                p1b, out_hbm.at[pl.ds(base, _CHUNK), pl.ds(d_word, d_pos)], ws)
            pltpu.async_copy(
                p2b, out_hbm.at[pl.ds(base, _CHUNK),
                                pl.ds(d_word + d_pos, d_pos)], ws)

        def wait_writes(s):
            wb, p1b, p2b, _, ws = bufs[s]
            rows = out_hbm.at[pl.ds(0, _CHUNK), pl.ds(0, d_word)]
            pltpu.make_async_copy(wb.at[:, pl.ds(0, d_word)], rows, ws).wait()
            pltpu.make_async_copy(
                p1b, out_hbm.at[pl.ds(0, _CHUNK), pl.ds(d_word, d_pos)],
                ws).wait()
            pltpu.make_async_copy(
                p2b, out_hbm.at[pl.ds(0, _CHUNK),
                                pl.ds(d_word + d_pos, d_pos)], ws).wait()

        n_pairs = n_chunks // 2
        fire_gathers(0, 0)

        def body(t, carry):
            j0 = 2 * t
            j1 = j0 + 1

            @pl.when(t > 0)
            def _():
                wait_writes(1)

            fire_gathers(j1, 1)
            wait_gathers(0)
            fire_writes(j0, 0)

            @pl.when(t < n_pairs - 1)
            def _():
                wait_writes(0)
                fire_gathers(j0 + 2, 0)

            wait_gathers(1)
            fire_writes(j1, 1)
            return carry

        lax.fori_loop(0, n_pairs, body, 0)
        wait_writes(0)
        wait_writes(1)

    return emb_kernel(word_i, pos1_i, pos2_i,
                      word_table, pos1_table, pos2_table)


def kernel(word, pos1, pos2, word_table, pos1_table, pos2_table):
    b, l = word.shape
    d_word = word_table.shape[1]
    d_pos = pos1_table.shape[1]
    n = b * l
    assert n % (_NUM_WORKERS * _CHUNK) == 0
    n_chunks = n // (_NUM_WORKERS * _CHUNK)
    assert n_chunks % 2 == 0

    shape = (_NUM_WORKERS, n_chunks, _CHUNK)
    word_i = word.reshape(shape).astype(jnp.int32)
    pos1_i = pos1.reshape(shape).astype(jnp.int32)
    pos2_i = pos2.reshape(shape).astype(jnp.int32)
    vocab = word_table.shape[0]
    word_table128 = jnp.concatenate(
        [word_table,
         jnp.zeros((vocab, _D_OUT - d_word), jnp.float32)], axis=1)

    out = _embed(word_i, pos1_i, pos2_i,
                 word_table128, pos1_table, pos2_table,
                 n_chunks, d_word, d_pos)
    return out[:, :d_word + 2 * d_pos].reshape(b, l, d_word + 2 * d_pos)


# speedup vs baseline: 1.8115x; 1.0003x over previous
import functools

import jax
import jax.numpy as jnp
from jax import lax
from jax.experimental import pallas as pl
from jax.experimental.pallas import tpu as pltpu
from jax.experimental.pallas import tpu_sc as plsc

_NUM_CORES = 2
_NUM_SUBCORES = 16
_NUM_WORKERS = _NUM_CORES * _NUM_SUBCORES
_CHUNK = 320
_D_OUT = 128


@functools.partial(jax.jit, static_argnames=("n_chunks", "d_word", "d_pos"))
def _embed(word_i, pos1_i, pos2_i, word_table, pos1_table, pos2_table,
           n_chunks, d_word, d_pos):
    n_total = _NUM_WORKERS * n_chunks * _CHUNK
    mesh = plsc.VectorSubcoreMesh(core_axis_name="c", subcore_axis_name="s")

    @functools.partial(
        pl.kernel,
        mesh=mesh,
        compiler_params=pltpu.CompilerParams(use_tc_tiling_on_sc=False),
        out_type=jax.ShapeDtypeStruct((n_total, _D_OUT), jnp.float32),
        scratch_types=[
            pltpu.VMEM((n_chunks, _CHUNK), jnp.int32),
            pltpu.VMEM((n_chunks, _CHUNK), jnp.int32),
            pltpu.VMEM((n_chunks, _CHUNK), jnp.int32),
            pltpu.VMEM((_CHUNK, _D_OUT), jnp.float32),
            pltpu.VMEM((_CHUNK, _D_OUT), jnp.float32),
            pltpu.VMEM((_CHUNK, 16), jnp.float32),
            pltpu.VMEM((_CHUNK, 16), jnp.float32),
            pltpu.VMEM((_CHUNK, 16), jnp.float32),
            pltpu.VMEM((_CHUNK, 16), jnp.float32),
            pltpu.SemaphoreType.DMA,
            pltpu.SemaphoreType.DMA,
            pltpu.SemaphoreType.DMA,
            pltpu.SemaphoreType.DMA,
        ],
    )
    def emb_kernel(w_hbm, p1_hbm, p2_hbm, wt_hbm, p1t_hbm, p2t_hbm, out_hbm,
                   widx, p1idx, p2idx, wbuf0, wbuf1, p1b0, p1b1, p2b0, p2b1,
                   gsem0, gsem1, wsem0, wsem1):
        wid = lax.axis_index("s") * _NUM_CORES + lax.axis_index("c")
        pltpu.sync_copy(w_hbm.at[wid], widx)
        pltpu.sync_copy(p1_hbm.at[wid], p1idx)
        pltpu.sync_copy(p2_hbm.at[wid], p2idx)
        base0 = wid * (n_chunks * _CHUNK)
        bufs = ((wbuf0, p1b0, p2b0, gsem0, wsem0),
                (wbuf1, p1b1, p2b1, gsem1, wsem1))

        def fire_gathers(j, s):
            wb, p1b, p2b, gs, _ = bufs[s]
            pltpu.async_copy(wt_hbm.at[widx.at[j]], wb, gs)
            pltpu.async_copy(p1t_hbm.at[p1idx.at[j]], p1b, gs)
            pltpu.async_copy(p2t_hbm.at[p2idx.at[j]], p2b, gs)

        def wait_gathers(s):
            wb, p1b, p2b, gs, _ = bufs[s]
            pltpu.make_async_copy(wt_hbm.at[pl.ds(0, _CHUNK)], wb, gs).wait()
            pltpu.make_async_copy(p1t_hbm.at[pl.ds(0, _CHUNK)], p1b, gs).wait()
            pltpu.make_async_copy(p2t_hbm.at[pl.ds(0, _CHUNK)], p2b, gs).wait()

        def fire_writes(j, s):
            wb, p1b, p2b, _, ws = bufs[s]
            base = base0 + j * _CHUNK
            rows = out_hbm.at[pl.ds(base, _CHUNK), pl.ds(0, d_word)]
            pltpu.async_copy(wb.at[:, pl.ds(0, d_word)], rows, ws)
            pltpu.async_copy(
                p1b, out_hbm.at[pl.ds(base, _CHUNK), pl.ds(d_word, d_pos)], ws)
            pltpu.async_copy(
                p2b, out_hbm.at[pl.ds(base, _CHUNK),
                                pl.ds(d_word + d_pos, d_pos)], ws)

        def wait_writes(s):
            wb, p1b, p2b, _, ws = bufs[s]
            rows = out_hbm.at[pl.ds(0, _CHUNK), pl.ds(0, d_word)]
            pltpu.make_async_copy(wb.at[:, pl.ds(0, d_word)], rows, ws).wait()
            pltpu.make_async_copy(
                p1b, out_hbm.at[pl.ds(0, _CHUNK), pl.ds(d_word, d_pos)],
                ws).wait()
            pltpu.make_async_copy(
                p2b, out_hbm.at[pl.ds(0, _CHUNK),
                                pl.ds(d_word + d_pos, d_pos)], ws).wait()

        n_pairs = n_chunks // 2
        fire_gathers(0, 0)

        def body(t, carry):
            j0 = 2 * t
            j1 = j0 + 1

            @pl.when(t > 0)
            def _():
                wait_writes(1)

            fire_gathers(j1, 1)
            wait_gathers(0)
            fire_writes(j0, 0)

            @pl.when(t < n_pairs - 1)
            def _():
                wait_writes(0)
                fire_gathers(j0 + 2, 0)

            wait_gathers(1)
            fire_writes(j1, 1)
            return carry

        lax.fori_loop(0, n_pairs, body, 0)
        wait_writes(0)
        wait_writes(1)

    return emb_kernel(word_i, pos1_i, pos2_i,
                      word_table, pos1_table, pos2_table)


def kernel(word, pos1, pos2, word_table, pos1_table, pos2_table):
    b, l = word.shape
    d_word = word_table.shape[1]
    d_pos = pos1_table.shape[1]
    n = b * l
    assert n % (_NUM_WORKERS * _CHUNK) == 0
    n_chunks = n // (_NUM_WORKERS * _CHUNK)
    assert n_chunks % 2 == 0

    shape = (_NUM_WORKERS, n_chunks, _CHUNK)
    word_i = word.reshape(shape).astype(jnp.int32)
    pos1_i = pos1.reshape(shape).astype(jnp.int32)
    pos2_i = pos2.reshape(shape).astype(jnp.int32)
    vocab = word_table.shape[0]
    word_table128 = jnp.concatenate(
        [word_table,
         jnp.zeros((vocab, _D_OUT - d_word), jnp.float32)], axis=1)

    out = _embed(word_i, pos1_i, pos2_i,
                 word_table128, pos1_table, pos2_table,
                 n_chunks, d_word, d_pos)
    return out[:, :d_word + 2 * d_pos].reshape(b, l, d_word + 2 * d_pos)


# speedup vs baseline: 1.8157x; 1.0023x over previous
import functools

import jax
import jax.numpy as jnp
from jax import lax
from jax.experimental import pallas as pl
from jax.experimental.pallas import tpu as pltpu
from jax.experimental.pallas import tpu_sc as plsc

_NUM_CORES = 2
_NUM_SUBCORES = 16
_NUM_WORKERS = _NUM_CORES * _NUM_SUBCORES
_CHUNK = 160
_D_OUT = 128


@functools.partial(jax.jit, static_argnames=("n_chunks", "d_word", "d_pos"))
def _embed(word_i, pos1_i, pos2_i, word_table, pos1_table, pos2_table,
           n_chunks, d_word, d_pos):
    n_total = _NUM_WORKERS * n_chunks * _CHUNK
    mesh = plsc.VectorSubcoreMesh(core_axis_name="c", subcore_axis_name="s")

    @functools.partial(
        pl.kernel,
        mesh=mesh,
        compiler_params=pltpu.CompilerParams(use_tc_tiling_on_sc=False),
        out_type=jax.ShapeDtypeStruct((n_total, _D_OUT), jnp.float32),
        scratch_types=[
            pltpu.VMEM((n_chunks, _CHUNK), jnp.int32),
            pltpu.VMEM((n_chunks, _CHUNK), jnp.int32),
            pltpu.VMEM((n_chunks, _CHUNK), jnp.int32),
            pltpu.VMEM((_CHUNK, _D_OUT), jnp.float32),
            pltpu.VMEM((_CHUNK, _D_OUT), jnp.float32),
            pltpu.VMEM((_CHUNK, _D_OUT), jnp.float32),
            pltpu.VMEM((_CHUNK, _D_OUT), jnp.float32),
            pltpu.VMEM((_CHUNK, 16), jnp.float32),
            pltpu.VMEM((_CHUNK, 16), jnp.float32),
            pltpu.VMEM((_CHUNK, 16), jnp.float32),
            pltpu.VMEM((_CHUNK, 16), jnp.float32),
            pltpu.VMEM((_CHUNK, 16), jnp.float32),
            pltpu.VMEM((_CHUNK, 16), jnp.float32),
            pltpu.VMEM((_CHUNK, 16), jnp.float32),
            pltpu.VMEM((_CHUNK, 16), jnp.float32),
            pltpu.SemaphoreType.DMA,
            pltpu.SemaphoreType.DMA,
            pltpu.SemaphoreType.DMA,
            pltpu.SemaphoreType.DMA,
            pltpu.SemaphoreType.DMA,
            pltpu.SemaphoreType.DMA,
            pltpu.SemaphoreType.DMA,
            pltpu.SemaphoreType.DMA,
        ],
    )
    def emb_kernel(w_hbm, p1_hbm, p2_hbm, wt_hbm, p1t_hbm, p2t_hbm, out_hbm,
                   widx, p1idx, p2idx, wbuf0, wbuf1, wbuf2, wbuf3,
                   p1b0, p1b1, p1b2, p1b3, p2b0, p2b1, p2b2, p2b3,
                   gsem0, gsem1, gsem2, gsem3, wsem0, wsem1, wsem2, wsem3):
        wid = lax.axis_index("s") * _NUM_CORES + lax.axis_index("c")
        pltpu.sync_copy(w_hbm.at[wid], widx)
        pltpu.sync_copy(p1_hbm.at[wid], p1idx)
        pltpu.sync_copy(p2_hbm.at[wid], p2idx)
        base0 = wid * (n_chunks * _CHUNK)
        bufs = ((wbuf0, p1b0, p2b0, gsem0, wsem0),
                (wbuf1, p1b1, p2b1, gsem1, wsem1),
                (wbuf2, p1b2, p2b2, gsem2, wsem2),
                (wbuf3, p1b3, p2b3, gsem3, wsem3))

        def fire_gathers(j, s):
            wb, p1b, p2b, gs, _ = bufs[s]
            pltpu.async_copy(wt_hbm.at[widx.at[j]], wb, gs)
            pltpu.async_copy(p1t_hbm.at[p1idx.at[j]], p1b, gs)
            pltpu.async_copy(p2t_hbm.at[p2idx.at[j]], p2b, gs)

        def wait_gathers(s):
            wb, p1b, p2b, gs, _ = bufs[s]
            pltpu.make_async_copy(wt_hbm.at[pl.ds(0, _CHUNK)], wb, gs).wait()
            pltpu.make_async_copy(p1t_hbm.at[pl.ds(0, _CHUNK)], p1b, gs).wait()
            pltpu.make_async_copy(p2t_hbm.at[pl.ds(0, _CHUNK)], p2b, gs).wait()

        def fire_writes(j, s):
            wb, p1b, p2b, _, ws = bufs[s]
            base = base0 + j * _CHUNK
            rows = out_hbm.at[pl.ds(base, _CHUNK), pl.ds(0, d_word)]
            pltpu.async_copy(wb.at[:, pl.ds(0, d_word)], rows, ws)
            pltpu.async_copy(
                p1b, out_hbm.at[pl.ds(base, _CHUNK), pl.ds(d_word, d_pos)], ws)
            pltpu.async_copy(
                p2b, out_hbm.at[pl.ds(base, _CHUNK),
                                pl.ds(d_word + d_pos, d_pos)], ws)

        def wait_writes(s):
            wb, p1b, p2b, _, ws = bufs[s]
            rows = out_hbm.at[pl.ds(0, _CHUNK), pl.ds(0, d_word)]
            pltpu.make_async_copy(wb.at[:, pl.ds(0, d_word)], rows, ws).wait()
            pltpu.make_async_copy(
                p1b, out_hbm.at[pl.ds(0, _CHUNK), pl.ds(d_word, d_pos)],
                ws).wait()
            pltpu.make_async_copy(
                p2b, out_hbm.at[pl.ds(0, _CHUNK),
                                pl.ds(d_word + d_pos, d_pos)], ws).wait()

        n_quads = n_chunks // 4
        fire_gathers(0, 0)
        fire_gathers(1, 1)

        def body(t, carry):
            for i in range(4):
                j = 4 * t + i
                s = i
                s2 = (i + 2) % 4

                @pl.when(j >= 2)
                def _():
                    wait_writes(s2)

                @pl.when(j + 2 < n_chunks)
                def _():
                    fire_gathers(j + 2, s2)

                wait_gathers(s)
                fire_writes(j, s)
            return carry

        lax.fori_loop(0, n_quads, body, 0)
        wait_writes(2)
        wait_writes(3)

    return emb_kernel(word_i, pos1_i, pos2_i,
                      word_table, pos1_table, pos2_table)


def kernel(word, pos1, pos2, word_table, pos1_table, pos2_table):
    b, l = word.shape
    d_word = word_table.shape[1]
    d_pos = pos1_table.shape[1]
    n = b * l
    assert n % (_NUM_WORKERS * _CHUNK) == 0
    n_chunks = n // (_NUM_WORKERS * _CHUNK)
    assert n_chunks % 2 == 0

    shape = (_NUM_WORKERS, n_chunks, _CHUNK)
    word_i = word.reshape(shape).astype(jnp.int32)
    pos1_i = pos1.reshape(shape).astype(jnp.int32)
    pos2_i = pos2.reshape(shape).astype(jnp.int32)
    vocab = word_table.shape[0]
    word_table128 = jnp.concatenate(
        [word_table,
         jnp.zeros((vocab, _D_OUT - d_word), jnp.float32)], axis=1)

    out = _embed(word_i, pos1_i, pos2_i,
                 word_table128, pos1_table, pos2_table,
                 n_chunks, d_word, d_pos)
    return out[:, :d_word + 2 * d_pos].reshape(b, l, d_word + 2 * d_pos)
